# baseline (speedup 1.0000x reference)
import functools

import jax
import jax.numpy as jnp
from jax import lax
from jax.experimental import pallas as pl
from jax.experimental.pallas import tpu as pltpu
from jax.experimental.pallas import tpu_sc as plsc

NUM_CLASSES = 100000
FEATURE_DIM = 64
BATCH = 16384

_NC, _NS, _L = 2, 16, 16
_NW = _NC * _NS
_BPW = BATCH // _NW
_GCHUNK = 128
_NG = _BPW // _GCHUNK


def _center_loss_body(feats_hbm, labels_hbm, centers_hbm, out_hbm,
                      idx_v, feats_v, rows_v, acc_v, gsem, fsem):
    wid = lax.axis_index("s") * _NC + lax.axis_index("c")

    pltpu.sync_copy(labels_hbm.at[wid], idx_v)
    gathers = [
        pltpu.make_async_copy(
            centers_hbm.at[idx_v.at[j]],
            rows_v.at[pl.ds(j * _GCHUNK, _GCHUNK)],
            gsem,
        )
        for j in range(_NG)
    ]
    for g in gathers:
        g.start()
    fcopy = pltpu.make_async_copy(feats_hbm.at[wid], feats_v, fsem)
    fcopy.start()
    fcopy.wait()
    for g in gathers:
        g.wait()

    zero = jnp.zeros((_L,), jnp.float32)

    def body(i, accs):
        new = []
        for j in range(FEATURE_DIM // _L):
            f = feats_v[i, pl.ds(j * _L, _L)]
            c = rows_v[i, pl.ds(j * _L, _L)]
            d = f - c
            new.append(accs[j] + d * d)
        return tuple(new)

    a0, a1, a2, a3 = lax.fori_loop(0, _BPW, body, (zero, zero, zero, zero))
    acc_v[...] = (a0 + a1) + (a2 + a3)
    pltpu.sync_copy(acc_v, out_hbm.at[wid])


@jax.jit
def _center_loss(features, labels, centers):
    mesh = plsc.VectorSubcoreMesh(core_axis_name="c", subcore_axis_name="s")
    partials = pl.kernel(
        _center_loss_body,
        out_type=jax.ShapeDtypeStruct((_NW, _L), jnp.float32),
        mesh=mesh,
        compiler_params=pltpu.CompilerParams(use_tc_tiling_on_sc=False),
        scratch_types=[
            pltpu.VMEM((_NG, _GCHUNK), jnp.int32),
            pltpu.VMEM((_BPW, FEATURE_DIM), jnp.float32),
            pltpu.VMEM((_BPW, FEATURE_DIM), jnp.float32),
            pltpu.VMEM((_L,), jnp.float32),
            pltpu.SemaphoreType.DMA,
            pltpu.SemaphoreType.DMA,
        ],
    )(features.reshape(_NW, _BPW, FEATURE_DIM),
      labels.astype(jnp.int32).reshape(_NW, _NG, _GCHUNK),
      centers)
    return jnp.sum(partials) / (BATCH * FEATURE_DIM)


def kernel(features, labels, centers):
    return _center_loss(features, labels, centers)
